# baseline (speedup 1.0000x reference)
import jax
import jax.numpy as jnp
from jax import lax
from jax.experimental import pallas as pl
from jax.experimental.pallas import tpu as pltpu
from jax.experimental.pallas import tpu_sc as plsc

BATCH = 4096
NFEAT = 39
NCAT = 26
NCVL = 13
EMB_D = 64
LANES = 16
FIELD_V = 100000

NCORES = 2
NSUB = 16
NWORK = NCORES * NSUB
EPW = BATCH // NWORK
CHUNK = 16
NCHUNK = EPW // CHUNK
SUBG = 4
EPS = CHUNK // SUBG
RPS = EPS * NCAT
ROW_W = EMB_D
VPAD = 100096
NTILE = VPAD // 128
NFULL = FIELD_V // 128
TAILC = FIELD_V - NFULL * 128
IPW = EPW * NCAT
XPW = EPW * NFEAT
NGRP = IPW // LANES


def _fm_body(x, emb, fc, bias, out,
             xv, idx_v, fcv, rows0, rows1, out_v, bias_v,
             sem_fc, sem0, sem1):
  wid = lax.axis_index("s") * NCORES + lax.axis_index("c")
  lane = lax.broadcasted_iota(jnp.int32, (LANES,), 0)

  pltpu.sync_copy(x.at[pl.ds(wid * XPW, XPW)], xv)
  pltpu.sync_copy(bias, bias_v)

  def build(k, carry):
    src, fpos = carry
    v = plsc.load_gather(xv, [src])
    idx_v[pl.ds(k * LANES, LANES)] = v.astype(jnp.int32)
    fnext = fpos + LANES
    wrap = fnext >= NCAT
    fnext = jnp.where(wrap, fnext - NCAT, fnext)
    src = src + LANES + jnp.where(wrap, NFEAT - NCAT, 0)
    return src, fnext

  lax.fori_loop(0, NGRP, build, (lane, lane))

  fc_descs = [
      pltpu.async_copy(fc.at[idx_v.at[pl.ds(g * RPS, RPS)]],
                       fcv.at[pl.ds(g * RPS, RPS)], sem_fc)
      for g in range(IPW // RPS)
  ]

  for t in range(2):
    rows_b, sem_b = (rows0, sem0) if t == 0 else (rows1, sem1)
    for j in range(SUBG):
      pltpu.async_copy(emb.at[idx_v.at[pl.ds(t * CHUNK * NCAT + j * RPS, RPS)]],
                       rows_b.at[pl.ds(j * RPS, RPS)], sem_b)

  for d in fc_descs:
    d.wait()
  bvec = bias_v[...]
  lane26 = lane * NCAT
  lane39 = lane * NFEAT
  for c in range(NCHUNK):
    acc = bvec
    for f in range(NCAT):
      v = plsc.load_gather(fcv, [lane26 + (c * CHUNK * NCAT + f)])
      if f >= NCAT - NCVL:
        cv = plsc.load_gather(
            xv, [lane39 + (c * CHUNK * NFEAT + NCAT + f - (NCAT - NCVL))])
        v = v * cv
      acc = acc + v
    out_v[pl.ds(c * CHUNK, CHUNK)] = acc

  def chunk_compute(t, rows_b):
    def elem(i, acc):
      base = i * NCAT
      zero = jnp.zeros((LANES,), jnp.float32)
      s0 = s1 = s2 = s3 = zero
      q0 = q1 = q2 = q3 = zero
      for f in range(NCAT):
        r = base + f
        e0 = rows_b[r, pl.ds(0, 16)]
        e1 = rows_b[r, pl.ds(16, 16)]
        e2 = rows_b[r, pl.ds(32, 16)]
        e3 = rows_b[r, pl.ds(48, 16)]
        s0 = s0 + e0
        s1 = s1 + e1
        s2 = s2 + e2
        s3 = s3 + e3
        q0 = q0 + e0 * e0
        q1 = q1 + e1 * e1
        q2 = q2 + e2 * e2
        q3 = q3 + e3 * e3
      d = (s0 * s0 - q0) + (s1 * s1 - q1) + (s2 * s2 - q2) + (s3 * s3 - q3)
      val = 0.5 * jnp.sum(d)
      return acc + jnp.where(lane == i, val, 0.0)

    off = pl.multiple_of(t * CHUNK, CHUNK)
    inter = lax.fori_loop(0, CHUNK, elem, jnp.zeros((LANES,), jnp.float32))
    z = out_v[pl.ds(off, CHUNK)] + inter
    out_v[pl.ds(off, CHUNK)] = 1.0 / (1.0 + jnp.exp(-z))

  def drain(t, rows_b, sem_b):
    for j in range(SUBG):
      pltpu.make_async_copy(
          emb.at[idx_v.at[pl.ds(t * CHUNK * NCAT + j * RPS, RPS)]],
          rows_b.at[pl.ds(j * RPS, RPS)], sem_b).wait()

  def step(g, carry):
    for b in range(2):
      t = g * 2 + b
      rows_b, sem_b = (rows0, sem0) if b == 0 else (rows1, sem1)
      drain(t, rows_b, sem_b)
      chunk_compute(t, rows_b)
      tn = t + 2
      for j in range(SUBG):
        pltpu.async_copy(
            emb.at[idx_v.at[pl.ds(tn * CHUNK * NCAT + j * RPS, RPS)]],
            rows_b.at[pl.ds(j * RPS, RPS)], sem_b)
    return carry

  lax.fori_loop(0, (NCHUNK - 2) // 2, step, 0)

  for t in (NCHUNK - 2, NCHUNK - 1):
    rows_b, sem_b = (rows0, sem0) if t % 2 == 0 else (rows1, sem1)
    drain(t, rows_b, sem_b)
    chunk_compute(t, rows_b)

  pltpu.sync_copy(out_v, out.at[pl.ds(wid * EPW, EPW)])


def _tr_body(embT, tailx, tbl, slab0, slab1, obuf0, obuf1,
             sem_i0, sem_i1, sem_o0, sem_o1):
  wid = lax.axis_index("s") * NCORES + lax.axis_index("c")
  lane = lax.broadcasted_iota(jnp.int32, (LANES,), 0)
  bufs = ((slab0, obuf0, sem_i0, sem_o0), (slab1, obuf1, sem_i1, sem_o1))

  def fire_in(j, b):
    slab, _, sem_i, _ = bufs[b]
    for i in range(EMB_D // 8):
      pltpu.async_copy(embT.at[pl.ds(8 * i, 8), pl.ds(j * 128, 128)],
                       slab.at[pl.ds(8 * i, 8)], sem_i)

  rowvs = [(lane + 16 * b2) // 2 for b2 in range(8)]
  colbase = (lane & 1) * EMB_D

  def transpose_tile(j, b, nrow):
    slab, obuf, sem_i, sem_o = bufs[b]
    for i in range(EMB_D // 8):
      pltpu.make_async_copy(embT.at[pl.ds(8 * i, 8), pl.ds(j * 128, 128)],
                            slab.at[pl.ds(8 * i, 8)], sem_i).wait()

    def body(d, carry):
      colv = colbase + d
      for b2 in range(8):
        v = slab[d, pl.ds(16 * b2, 16)]
        plsc.store_scatter(obuf, [rowvs[b2], colv], v)
      return carry

    lax.fori_loop(0, EMB_D, body, 0)
    pltpu.async_copy(obuf.at[pl.ds(0, nrow)], tbl.at[pl.ds(j * 64, nrow)],
                     sem_o)

  def drain_out(j, b, nrow):
    _, obuf, _, sem_o = bufs[b]
    pltpu.make_async_copy(obuf.at[pl.ds(0, nrow)],
                          tbl.at[pl.ds(j * 64, nrow)], sem_o).wait()

  fire_in(wid, 0)
  fire_in(wid + 32, 1)
  for b in range(2):
    transpose_tile(wid + 32 * b, b, 64)
    fire_in(wid + 32 * (b + 2), b)

  def step(g, carry):
    for b in range(2):
      n = g * 2 + b
      j = wid + 32 * n
      drain_out(j - 64, b, 64)
      transpose_tile(j, b, 64)
      @pl.when(n + 2 < 24)
      def _():
        fire_in(j + 64, b)
    return carry

  lax.fori_loop(1, 12, step, 0)

  for b in range(2):
    n = 22 + b
    drain_out(wid + 32 * n, b, 64)

  @pl.when(wid <= 12)
  def _():
    j = wid + 768
    fire_in(j, 0)
    transpose_tile(j, 0, 64)
    drain_out(j, 0, 64)

  @pl.when(wid == 13)
  def _():
    pltpu.sync_copy(tailx, obuf0.at[pl.ds(0, TAILC // 2)])
    pltpu.sync_copy(obuf0.at[pl.ds(0, TAILC // 2)],
                    tbl.at[pl.ds(NFULL * 64, TAILC // 2)])


_FM_CALL = None
_TR_CALL = None


def _get_tr_call():
  global _TR_CALL
  if _TR_CALL is None:
    mesh = plsc.VectorSubcoreMesh(core_axis_name="c", subcore_axis_name="s",
                                  num_cores=NCORES, num_subcores=NSUB)
    scratch = [
        pltpu.VMEM((EMB_D, 128), jnp.float32),
        pltpu.VMEM((EMB_D, 128), jnp.float32),
        pltpu.VMEM((EMB_D, 128), jnp.float32),
        pltpu.VMEM((EMB_D, 128), jnp.float32),
        pltpu.SemaphoreType.DMA,
        pltpu.SemaphoreType.DMA,
        pltpu.SemaphoreType.DMA,
        pltpu.SemaphoreType.DMA,
    ]
    _TR_CALL = pl.kernel(
        _tr_body,
        out_type=jax.ShapeDtypeStruct((VPAD // 2, 128), jnp.float32),
        mesh=mesh,
        scratch_types=scratch,
        compiler_params=pltpu.CompilerParams(needs_layout_passes=False,
                                             use_tc_tiling_on_sc=True),
    )
  return _TR_CALL


def _get_fm_call():
  global _FM_CALL
  if _FM_CALL is None:
    mesh = plsc.VectorSubcoreMesh(core_axis_name="c", subcore_axis_name="s",
                                  num_cores=NCORES, num_subcores=NSUB)
    scratch = [
        pltpu.VMEM((XPW,), jnp.float32),
        pltpu.VMEM((IPW,), jnp.int32),
        pltpu.VMEM((IPW,), jnp.float32),
        pltpu.VMEM((CHUNK * NCAT, ROW_W), jnp.float32),
        pltpu.VMEM((CHUNK * NCAT, ROW_W), jnp.float32),
        pltpu.VMEM((EPW,), jnp.float32),
        pltpu.VMEM((LANES,), jnp.float32),
        pltpu.SemaphoreType.DMA,
        pltpu.SemaphoreType.DMA,
        pltpu.SemaphoreType.DMA,
    ]
    _FM_CALL = pl.kernel(
        _fm_body,
        out_type=jax.ShapeDtypeStruct((BATCH,), jnp.float32),
        mesh=mesh,
        scratch_types=scratch,
        compiler_params=pltpu.CompilerParams(needs_layout_passes=False,
                                             use_tc_tiling_on_sc=False),
    )
  return _FM_CALL


def kernel(x, emb_weight, fc_weight, bias):
  x_flat = x.reshape(-1)
  tailx = emb_weight[NFULL * 128:, :].reshape(TAILC // 2, 2 * EMB_D)
  tbl = _get_tr_call()(emb_weight.T, tailx)
  emb_lin = tbl.reshape(VPAD, EMB_D)
  fc_flat = fc_weight.reshape(-1)
  bias16 = jnp.broadcast_to(bias.astype(jnp.float32), (LANES,))
  return _get_fm_call()(x_flat, emb_lin, fc_flat, bias16)


# speedup vs baseline: 1.0461x; 1.0461x over previous
import jax
import jax.numpy as jnp
from jax import lax
from jax.experimental import pallas as pl
from jax.experimental.pallas import tpu as pltpu
from jax.experimental.pallas import tpu_sc as plsc

BATCH = 4096
NFEAT = 39
NCAT = 26
NCVL = 13
EMB_D = 64
LANES = 16
FIELD_V = 100000

NCORES = 2
NSUB = 16
NWORK = NCORES * NSUB
EPW = BATCH // NWORK
CHUNK = 16
NCHUNK = EPW // CHUNK
SUBG = 4
EPS = CHUNK // SUBG
RPS = EPS * NCAT
ROW_W = EMB_D
TRB = 256
HALF = 50176
NBLK = HALF // TRB
VPAD = 2 * HALF
IPW = EPW * NCAT
XPW = EPW * NFEAT
NGRP = IPW // LANES


def _fm_body(x, emb, fc, bias, out,
             xv, idx_v, idx_r, fcv, rows0, rows1, out_v, bias_v,
             sem_fc, sem0, sem1):
  wid = lax.axis_index("s") * NCORES + lax.axis_index("c")
  lane = lax.broadcasted_iota(jnp.int32, (LANES,), 0)

  pltpu.sync_copy(x.at[pl.ds(wid * XPW, XPW)], xv)
  pltpu.sync_copy(bias, bias_v)

  def build(k, carry):
    src, fpos = carry
    v = plsc.load_gather(xv, [src])
    vi = v.astype(jnp.int32)
    idx_r[pl.ds(k * LANES, LANES)] = vi
    vi = vi + vi - jnp.where(vi >= HALF, 2 * HALF - 1, 0)
    idx_v[pl.ds(k * LANES, LANES)] = vi
    fnext = fpos + LANES
    wrap = fnext >= NCAT
    fnext = jnp.where(wrap, fnext - NCAT, fnext)
    src = src + LANES + jnp.where(wrap, NFEAT - NCAT, 0)
    return src, fnext

  lax.fori_loop(0, NGRP, build, (lane, lane))

  fc_descs = [
      pltpu.async_copy(fc.at[idx_r.at[pl.ds(g * RPS, RPS)]],
                       fcv.at[pl.ds(g * RPS, RPS)], sem_fc)
      for g in range(IPW // RPS)
  ]

  for t in range(2):
    rows_b, sem_b = (rows0, sem0) if t == 0 else (rows1, sem1)
    for j in range(SUBG):
      pltpu.async_copy(emb.at[idx_v.at[pl.ds(t * CHUNK * NCAT + j * RPS, RPS)]],
                       rows_b.at[pl.ds(j * RPS, RPS)], sem_b)

  for d in fc_descs:
    d.wait()
  bvec = bias_v[...]
  lane26 = lane * NCAT
  lane39 = lane * NFEAT
  for c in range(NCHUNK):
    acc = bvec
    for f in range(NCAT):
      v = plsc.load_gather(fcv, [lane26 + (c * CHUNK * NCAT + f)])
      if f >= NCAT - NCVL:
        cv = plsc.load_gather(
            xv, [lane39 + (c * CHUNK * NFEAT + NCAT + f - (NCAT - NCVL))])
        v = v * cv
      acc = acc + v
    out_v[pl.ds(c * CHUNK, CHUNK)] = acc

  def chunk_compute(t, rows_b):
    def elem(i, acc):
      base = i * NCAT
      zero = jnp.zeros((LANES,), jnp.float32)
      s0 = s1 = s2 = s3 = zero
      q0 = q1 = q2 = q3 = zero
      for f in range(NCAT):
        r = base + f
        e0 = rows_b[r, pl.ds(0, 16)]
        e1 = rows_b[r, pl.ds(16, 16)]
        e2 = rows_b[r, pl.ds(32, 16)]
        e3 = rows_b[r, pl.ds(48, 16)]
        s0 = s0 + e0
        s1 = s1 + e1
        s2 = s2 + e2
        s3 = s3 + e3
        q0 = q0 + e0 * e0
        q1 = q1 + e1 * e1
        q2 = q2 + e2 * e2
        q3 = q3 + e3 * e3
      d = (s0 * s0 - q0) + (s1 * s1 - q1) + (s2 * s2 - q2) + (s3 * s3 - q3)
      val = 0.5 * jnp.sum(d)
      return acc + jnp.where(lane == i, val, 0.0)

    off = pl.multiple_of(t * CHUNK, CHUNK)
    inter = lax.fori_loop(0, CHUNK, elem, jnp.zeros((LANES,), jnp.float32))
    z = out_v[pl.ds(off, CHUNK)] + inter
    out_v[pl.ds(off, CHUNK)] = 1.0 / (1.0 + jnp.exp(-z))

  def drain(t, rows_b, sem_b):
    for j in range(SUBG):
      pltpu.make_async_copy(
          emb.at[idx_v.at[pl.ds(t * CHUNK * NCAT + j * RPS, RPS)]],
          rows_b.at[pl.ds(j * RPS, RPS)], sem_b).wait()

  def step(g, carry):
    for b in range(2):
      t = g * 2 + b
      rows_b, sem_b = (rows0, sem0) if b == 0 else (rows1, sem1)
      drain(t, rows_b, sem_b)
      chunk_compute(t, rows_b)
      tn = t + 2
      for j in range(SUBG):
        pltpu.async_copy(
            emb.at[idx_v.at[pl.ds(tn * CHUNK * NCAT + j * RPS, RPS)]],
            rows_b.at[pl.ds(j * RPS, RPS)], sem_b)
    return carry

  lax.fori_loop(0, (NCHUNK - 2) // 2, step, 0)

  for t in (NCHUNK - 2, NCHUNK - 1):
    rows_b, sem_b = (rows0, sem0) if t % 2 == 0 else (rows1, sem1)
    drain(t, rows_b, sem_b)
    chunk_compute(t, rows_b)

  pltpu.sync_copy(out_v, out.at[pl.ds(wid * EPW, EPW)])


def _tc_transpose_body(xa_ref, xb_ref, o_ref):
  o_ref[...] = jnp.concatenate([xa_ref[...].T, xb_ref[...].T], axis=1)


_TC_TR = None


def _get_tc_tr():
  global _TC_TR
  if _TC_TR is None:
    _TC_TR = pl.pallas_call(
        _tc_transpose_body,
        grid=(NBLK,),
        in_specs=[pl.BlockSpec((EMB_D, TRB), lambda i: (0, i)),
                  pl.BlockSpec((EMB_D, TRB), lambda i: (0, i + NBLK))],
        out_specs=pl.BlockSpec((TRB, 2 * EMB_D), lambda i: (i, 0)),
        out_shape=jax.ShapeDtypeStruct((HALF, 2 * EMB_D), jnp.float32),
    )
  return _TC_TR


_FM_CALL = None
def _get_fm_call():
  global _FM_CALL
  if _FM_CALL is None:
    mesh = plsc.VectorSubcoreMesh(core_axis_name="c", subcore_axis_name="s",
                                  num_cores=NCORES, num_subcores=NSUB)
    scratch = [
        pltpu.VMEM((XPW,), jnp.float32),
        pltpu.VMEM((IPW,), jnp.int32),
        pltpu.VMEM((IPW,), jnp.int32),
        pltpu.VMEM((IPW,), jnp.float32),
        pltpu.VMEM((CHUNK * NCAT, ROW_W), jnp.float32),
        pltpu.VMEM((CHUNK * NCAT, ROW_W), jnp.float32),
        pltpu.VMEM((EPW,), jnp.float32),
        pltpu.VMEM((LANES,), jnp.float32),
        pltpu.SemaphoreType.DMA,
        pltpu.SemaphoreType.DMA,
        pltpu.SemaphoreType.DMA,
    ]
    _FM_CALL = pl.kernel(
        _fm_body,
        out_type=jax.ShapeDtypeStruct((BATCH,), jnp.float32),
        mesh=mesh,
        scratch_types=scratch,
        compiler_params=pltpu.CompilerParams(needs_layout_passes=False,
                                             use_tc_tiling_on_sc=False),
    )
  return _FM_CALL


def kernel(x, emb_weight, fc_weight, bias):
  x_flat = x.reshape(-1)
  embT = emb_weight.T
  tbl = _get_tc_tr()(embT, embT)
  emb_lin = tbl.reshape(VPAD, EMB_D)
  fc_flat = fc_weight.reshape(-1)
  bias16 = jnp.broadcast_to(bias.astype(jnp.float32), (LANES,))
  return _get_fm_call()(x_flat, emb_lin, fc_flat, bias16)


# speedup vs baseline: 1.6932x; 1.6185x over previous
import jax
import jax.numpy as jnp
from jax import lax
from jax.experimental import pallas as pl
from jax.experimental.pallas import tpu as pltpu
from jax.experimental.pallas import tpu_sc as plsc

BATCH = 4096
NFEAT = 39
NCAT = 26
NCVL = 13
EMB_D = 64
LANES = 16

NCORES = 2
NSUB = 16
NWORK = NCORES * NSUB
EPW = BATCH // NWORK
CHUNK = 16
NCHUNK = EPW // CHUNK
SUBG = 4
EPS = CHUNK // SUBG
RPS = EPS * NCAT
ROW_W = EMB_D
IPW = EPW * NCAT
XPW = EPW * NFEAT
NGRP = IPW // LANES


def _fm_body(x, emb, fc, bias, out,
             xv, idx_v, fcv, rows0, rows1, out_v, bias_v,
             sem_fc, sem0, sem1):
  wid = lax.axis_index("s") * NCORES + lax.axis_index("c")
  lane = lax.broadcasted_iota(jnp.int32, (LANES,), 0)

  pltpu.sync_copy(x.at[pl.ds(wid * XPW, XPW)], xv)
  pltpu.sync_copy(bias, bias_v)

  def build(k, carry):
    src, fpos = carry
    v = plsc.load_gather(xv, [src])
    idx_v[pl.ds(k * LANES, LANES)] = v.astype(jnp.int32)
    fnext = fpos + LANES
    wrap = fnext >= NCAT
    fnext = jnp.where(wrap, fnext - NCAT, fnext)
    src = src + LANES + jnp.where(wrap, NFEAT - NCAT, 0)
    return src, fnext

  lax.fori_loop(0, NGRP, build, (lane, lane))

  fc_descs = [
      pltpu.async_copy(fc.at[idx_v.at[pl.ds(g * RPS, RPS)]],
                       fcv.at[pl.ds(g * RPS, RPS)], sem_fc)
      for g in range(IPW // RPS)
  ]

  for t in range(2):
    rows_b, sem_b = (rows0, sem0) if t == 0 else (rows1, sem1)
    for j in range(SUBG):
      pltpu.async_copy(emb.at[idx_v.at[pl.ds(t * CHUNK * NCAT + j * RPS, RPS)]],
                       rows_b.at[pl.ds(j * RPS, RPS)], sem_b)

  for d in fc_descs:
    d.wait()
  bvec = bias_v[...]
  lane26 = lane * NCAT
  lane39 = lane * NFEAT
  for c in range(NCHUNK):
    acc = bvec
    for f in range(NCAT):
      v = plsc.load_gather(fcv, [lane26 + (c * CHUNK * NCAT + f)])
      if f >= NCAT - NCVL:
        cv = plsc.load_gather(
            xv, [lane39 + (c * CHUNK * NFEAT + NCAT + f - (NCAT - NCVL))])
        v = v * cv
      acc = acc + v
    out_v[pl.ds(c * CHUNK, CHUNK)] = acc

  def chunk_compute(t, rows_b):
    def elem(i, acc):
      base = i * NCAT
      zero = jnp.zeros((LANES,), jnp.float32)
      s0 = s1 = s2 = s3 = zero
      q0 = q1 = q2 = q3 = zero
      for f in range(NCAT):
        r = base + f
        e0 = rows_b[r, pl.ds(0, 16)]
        e1 = rows_b[r, pl.ds(16, 16)]
        e2 = rows_b[r, pl.ds(32, 16)]
        e3 = rows_b[r, pl.ds(48, 16)]
        s0 = s0 + e0
        s1 = s1 + e1
        s2 = s2 + e2
        s3 = s3 + e3
        q0 = q0 + e0 * e0
        q1 = q1 + e1 * e1
        q2 = q2 + e2 * e2
        q3 = q3 + e3 * e3
      d = (s0 * s0 - q0) + (s1 * s1 - q1) + (s2 * s2 - q2) + (s3 * s3 - q3)
      val = 0.5 * jnp.sum(d)
      return acc + jnp.where(lane == i, val, 0.0)

    off = pl.multiple_of(t * CHUNK, CHUNK)
    inter = lax.fori_loop(0, CHUNK, elem, jnp.zeros((LANES,), jnp.float32))
    z = out_v[pl.ds(off, CHUNK)] + inter
    out_v[pl.ds(off, CHUNK)] = 1.0 / (1.0 + jnp.exp(-z))

  def drain(t, rows_b, sem_b):
    for j in range(SUBG):
      pltpu.make_async_copy(
          emb.at[idx_v.at[pl.ds(t * CHUNK * NCAT + j * RPS, RPS)]],
          rows_b.at[pl.ds(j * RPS, RPS)], sem_b).wait()

  def step(g, carry):
    for b in range(2):
      t = g * 2 + b
      rows_b, sem_b = (rows0, sem0) if b == 0 else (rows1, sem1)
      drain(t, rows_b, sem_b)
      chunk_compute(t, rows_b)
      tn = t + 2
      for j in range(SUBG):
        pltpu.async_copy(
            emb.at[idx_v.at[pl.ds(tn * CHUNK * NCAT + j * RPS, RPS)]],
            rows_b.at[pl.ds(j * RPS, RPS)], sem_b)
    return carry

  lax.fori_loop(0, (NCHUNK - 2) // 2, step, 0)

  for t in (NCHUNK - 2, NCHUNK - 1):
    rows_b, sem_b = (rows0, sem0) if t % 2 == 0 else (rows1, sem1)
    drain(t, rows_b, sem_b)
    chunk_compute(t, rows_b)

  pltpu.sync_copy(out_v, out.at[pl.ds(wid * EPW, EPW)])


_FM_CALL = None


def _get_fm_call():
  global _FM_CALL
  if _FM_CALL is None:
    mesh = plsc.VectorSubcoreMesh(core_axis_name="c", subcore_axis_name="s",
                                  num_cores=NCORES, num_subcores=NSUB)
    scratch = [
        pltpu.VMEM((XPW,), jnp.float32),
        pltpu.VMEM((IPW,), jnp.int32),
        pltpu.VMEM((IPW,), jnp.float32),
        pltpu.VMEM((CHUNK * NCAT, ROW_W), jnp.float32),
        pltpu.VMEM((CHUNK * NCAT, ROW_W), jnp.float32),
        pltpu.VMEM((EPW,), jnp.float32),
        pltpu.VMEM((LANES,), jnp.float32),
        pltpu.SemaphoreType.DMA,
        pltpu.SemaphoreType.DMA,
        pltpu.SemaphoreType.DMA,
    ]
    _FM_CALL = pl.kernel(
        _fm_body,
        out_type=jax.ShapeDtypeStruct((BATCH,), jnp.float32),
        mesh=mesh,
        scratch_types=scratch,
        compiler_params=pltpu.CompilerParams(needs_layout_passes=False,
                                             use_tc_tiling_on_sc=False),
    )
  return _FM_CALL


def kernel(x, emb_weight, fc_weight, bias):
  x_flat = x.reshape(-1)
  fc_flat = fc_weight.reshape(-1)
  bias16 = jnp.broadcast_to(bias.astype(jnp.float32), (LANES,))
  return _get_fm_call()(x_flat, emb_weight, fc_flat, bias16)
